# baseline (speedup 1.0000x reference)
import functools

import jax
import jax.numpy as jnp
from jax import lax
from jax.experimental import pallas as pl
from jax.experimental.pallas import tpu as pltpu
from jax.experimental.pallas import tpu_sc as plsc

_INFO = plsc.get_sparse_core_info()
_NC = _INFO.num_cores
_NS = _INFO.num_subcores
_NW = _NC * _NS
_CHUNK = 128


_BI = 512
_NMAIN = 976
_H = _BI * _NMAIN


@functools.lru_cache(maxsize=None)
def _make_repack(V, D):
    n_tail = (V - 2 * _H + _BI - 1) // _BI
    grid = _NMAIN + n_tail

    def body(lo_ref, hi_ref, out_ref):
        eye = jnp.eye(D, dtype=jnp.float32)
        lo = lax.dot_general(
            lo_ref[...], eye, (((0,), (0,)), ((), ())),
            preferred_element_type=jnp.float32,
        )
        hi = lax.dot_general(
            hi_ref[...], eye, (((0,), (0,)), ((), ())),
            preferred_element_type=jnp.float32,
        )
        out_ref[:, 0:D] = lo
        out_ref[:, D : 2 * D] = hi

    return pl.pallas_call(
        body,
        grid=(grid,),
        in_specs=[
            pl.BlockSpec(
                (D, _BI),
                lambda g: (0, jnp.where(g < _NMAIN, g, g + _NMAIN)),
            ),
            pl.BlockSpec((D, _BI), lambda g: (0, g + _NMAIN)),
        ],
        out_specs=pl.BlockSpec((_BI, 2 * D), lambda g: (g, 0)),
        out_shape=jax.ShapeDtypeStruct((grid * _BI, 2 * D), jnp.float32),
    )


@functools.lru_cache(maxsize=None)
def _make_lookup(V, D, B):
    b_per_w = B // _NW
    n_chunks = b_per_w // _CHUNK
    mesh = plsc.VectorSubcoreMesh(core_axis_name="c", subcore_axis_name="s")

    @functools.partial(
        pl.kernel,
        mesh=mesh,
        out_type=jax.ShapeDtypeStruct((B * D,), jnp.float32),
        scratch_types=[
            pltpu.VMEM((b_per_w,), jnp.int32),
            pltpu.VMEM((n_chunks, _CHUNK), jnp.int32),
            pltpu.VMEM((b_per_w, 2 * D), jnp.float32),
            pltpu.VMEM((b_per_w * D,), jnp.float32),
            pltpu.SemaphoreType.DMA,
        ],
    )
    def lookup(packed_hbm, idx_hbm, out_hbm, idx_v, p_v, pairs_v, rows_v, sem):
        wid = lax.axis_index("s") * _NC + lax.axis_index("c")
        base = wid * b_per_w
        pltpu.sync_copy(idx_hbm.at[pl.ds(base, b_per_w)], idx_v)

        for r in range(n_chunks):

            def mkp(g, _, r=r):
                vec = idx_v[pl.ds(r * _CHUNK + g * 16, 16)]
                p_v[r, pl.ds(g * 16, 16)] = jnp.where(vec >= _H, vec - _H, vec)
                return 0

            lax.fori_loop(0, _CHUNK // 16, mkp, 0)

        copies = [
            pltpu.async_copy(
                packed_hbm.at[p_v.at[r]],
                pairs_v.at[pl.ds(r * _CHUNK, _CHUNK)],
                sem,
            )
            for r in range(n_chunks)
        ]
        for c in copies:
            c.wait()

        def sel(g, _):
            vec = idx_v[pl.ds(g * 16, 16)]
            for j in range(16):
                k = g * 16 + j
                half = (vec[j] >= _H).astype(jnp.int32) * D
                for t in range(D // 16):
                    rows_v[pl.ds(k * D + t * 16, 16)] = pairs_v[
                        k, pl.ds(half + t * 16, 16)
                    ]
            return 0

        lax.fori_loop(0, b_per_w // 16, sel, 0)
        pltpu.sync_copy(rows_v, out_hbm.at[pl.ds(base * D, b_per_w * D)])

    return lookup


def kernel(x, age_embedding_weight):
    (B,) = x.shape
    V, D = age_embedding_weight.shape
    wt = age_embedding_weight.T
    packed = _make_repack(V, D)(wt, wt)
    flat = _make_lookup(V, D, B)(packed, x.astype(jnp.int32))
    return flat.reshape(B, D)


# speedup vs baseline: 2.0890x; 2.0890x over previous
import functools

import jax
import jax.numpy as jnp
from jax import lax
from jax.experimental import pallas as pl
from jax.experimental.pallas import tpu as pltpu
from jax.experimental.pallas import tpu_sc as plsc

_INFO = plsc.get_sparse_core_info()
_NC = _INFO.num_cores
_NS = _INFO.num_subcores
_NW = _NC * _NS
_CHUNK = 128


_BI = 2048
_NMAIN = 244
_H = _BI * _NMAIN


@functools.lru_cache(maxsize=None)
def _make_repack(V, D):
    n_tail = (V - 2 * _H + _BI - 1) // _BI
    grid = _NMAIN + n_tail

    def body(lo_ref, hi_ref, out_ref):
        out_ref[:, 0:D] = lo_ref[...].T
        out_ref[:, D : 2 * D] = hi_ref[...].T

    return pl.pallas_call(
        body,
        grid=(grid,),
        in_specs=[
            pl.BlockSpec(
                (D, _BI),
                lambda g: (0, jnp.where(g < _NMAIN, g, g + _NMAIN)),
            ),
            pl.BlockSpec((D, _BI), lambda g: (0, g + _NMAIN)),
        ],
        out_specs=pl.BlockSpec((_BI, 2 * D), lambda g: (g, 0)),
        out_shape=jax.ShapeDtypeStruct((grid * _BI, 2 * D), jnp.float32),
    )


@functools.lru_cache(maxsize=None)
def _make_lookup(V, D, B):
    b_per_w = B // _NW
    n_chunks = b_per_w // _CHUNK
    mesh = plsc.VectorSubcoreMesh(core_axis_name="c", subcore_axis_name="s")

    @functools.partial(
        pl.kernel,
        mesh=mesh,
        out_type=jax.ShapeDtypeStruct((B * D,), jnp.float32),
        scratch_types=[
            pltpu.VMEM((b_per_w,), jnp.int32),
            pltpu.VMEM((n_chunks, _CHUNK), jnp.int32),
            pltpu.VMEM((b_per_w, 2 * D), jnp.float32),
            pltpu.VMEM((b_per_w * D,), jnp.float32),
            pltpu.SemaphoreType.DMA,
        ],
    )
    def lookup(packed_hbm, idx_hbm, out_hbm, idx_v, p_v, pairs_v, rows_v, sem):
        wid = lax.axis_index("s") * _NC + lax.axis_index("c")
        base = wid * b_per_w
        pltpu.sync_copy(idx_hbm.at[pl.ds(base, b_per_w)], idx_v)

        for r in range(n_chunks):

            def mkp(g, _, r=r):
                vec = idx_v[pl.ds(r * _CHUNK + g * 16, 16)]
                p_v[r, pl.ds(g * 16, 16)] = jnp.where(vec >= _H, vec - _H, vec)
                return 0

            lax.fori_loop(0, _CHUNK // 16, mkp, 0)

        copies = [
            pltpu.async_copy(
                packed_hbm.at[p_v.at[r]],
                pairs_v.at[pl.ds(r * _CHUNK, _CHUNK)],
                sem,
            )
            for r in range(n_chunks)
        ]
        for c in copies:
            c.wait()

        def sel(g, _):
            vec = idx_v[pl.ds(g * 16, 16)]
            for j in range(16):
                k = g * 16 + j
                half = (vec[j] >= _H).astype(jnp.int32) * D
                for t in range(D // 16):
                    rows_v[pl.ds(k * D + t * 16, 16)] = pairs_v[
                        k, pl.ds(half + t * 16, 16)
                    ]
            return 0

        lax.fori_loop(0, b_per_w // 16, sel, 0)
        pltpu.sync_copy(rows_v, out_hbm.at[pl.ds(base * D, b_per_w * D)])

    return lookup


def kernel(x, age_embedding_weight):
    (B,) = x.shape
    V, D = age_embedding_weight.shape
    wt = age_embedding_weight.T
    packed = _make_repack(V, D)(wt, wt)
    flat = _make_lookup(V, D, B)(packed, x.astype(jnp.int32))
    return flat.reshape(B, D)


# speedup vs baseline: 2.5663x; 1.2285x over previous
import functools

import jax
import jax.numpy as jnp
from jax import lax
from jax.experimental import pallas as pl
from jax.experimental.pallas import tpu as pltpu
from jax.experimental.pallas import tpu_sc as plsc

_INFO = plsc.get_sparse_core_info()
_NC = _INFO.num_cores
_NS = _INFO.num_subcores
_NW = _NC * _NS
_CHUNK = 128


_BI = 4096
_NMAIN = 122
_H = _BI * _NMAIN


@functools.lru_cache(maxsize=None)
def _make_repack(V, D):
    n_tail = (V - 2 * _H + _BI - 1) // _BI
    grid = _NMAIN + n_tail

    def body(lo_ref, hi_ref, out_ref):
        out_ref[...] = jnp.concatenate(
            [lo_ref[...].T, hi_ref[...].T], axis=1
        )

    return pl.pallas_call(
        body,
        grid=(grid,),
        in_specs=[
            pl.BlockSpec(
                (D, _BI),
                lambda g: (0, jnp.where(g < _NMAIN, g, g + _NMAIN)),
            ),
            pl.BlockSpec((D, _BI), lambda g: (0, g + _NMAIN)),
        ],
        out_specs=pl.BlockSpec((_BI, 2 * D), lambda g: (g, 0)),
        out_shape=jax.ShapeDtypeStruct((grid * _BI, 2 * D), jnp.float32),
    )


@functools.lru_cache(maxsize=None)
def _make_lookup(V, D, B):
    b_per_w = B // _NW
    n_chunks = b_per_w // _CHUNK
    mesh = plsc.VectorSubcoreMesh(core_axis_name="c", subcore_axis_name="s")

    @functools.partial(
        pl.kernel,
        mesh=mesh,
        out_type=jax.ShapeDtypeStruct((B * D,), jnp.float32),
        scratch_types=[
            pltpu.VMEM((b_per_w,), jnp.int32),
            pltpu.VMEM((n_chunks, _CHUNK), jnp.int32),
            pltpu.VMEM((b_per_w, 2 * D), jnp.float32),
            pltpu.VMEM((b_per_w * D,), jnp.float32),
            pltpu.SemaphoreType.DMA,
        ],
    )
    def lookup(packed_hbm, idx_hbm, out_hbm, idx_v, p_v, pairs_v, rows_v, sem):
        wid = lax.axis_index("s") * _NC + lax.axis_index("c")
        base = wid * b_per_w
        pltpu.sync_copy(idx_hbm.at[pl.ds(base, b_per_w)], idx_v)

        for r in range(n_chunks):

            def mkp(g, _, r=r):
                vec = idx_v[pl.ds(r * _CHUNK + g * 16, 16)]
                p_v[r, pl.ds(g * 16, 16)] = jnp.where(vec >= _H, vec - _H, vec)
                return 0

            lax.fori_loop(0, _CHUNK // 16, mkp, 0)

        copies = [
            pltpu.async_copy(
                packed_hbm.at[p_v.at[r]],
                pairs_v.at[pl.ds(r * _CHUNK, _CHUNK)],
                sem,
            )
            for r in range(n_chunks)
        ]
        for c in copies:
            c.wait()

        def sel(g, _):
            vec = idx_v[pl.ds(g * 16, 16)]
            for j in range(16):
                k = g * 16 + j
                half = (vec[j] >= _H).astype(jnp.int32) * D
                for t in range(D // 16):
                    rows_v[pl.ds(k * D + t * 16, 16)] = pairs_v[
                        k, pl.ds(half + t * 16, 16)
                    ]
            return 0

        lax.fori_loop(0, b_per_w // 16, sel, 0)
        pltpu.sync_copy(rows_v, out_hbm.at[pl.ds(base * D, b_per_w * D)])

    return lookup


def kernel(x, age_embedding_weight):
    (B,) = x.shape
    V, D = age_embedding_weight.shape
    wt = age_embedding_weight.T
    packed = _make_repack(V, D)(wt, wt)
    flat = _make_lookup(V, D, B)(packed, x.astype(jnp.int32))
    return flat.reshape(B, D)


# speedup vs baseline: 2.8624x; 1.1154x over previous
import functools

import jax
import jax.numpy as jnp
from jax import lax
from jax.experimental import pallas as pl
from jax.experimental.pallas import tpu as pltpu
from jax.experimental.pallas import tpu_sc as plsc

_INFO = plsc.get_sparse_core_info()
_NC = _INFO.num_cores
_NS = _INFO.num_subcores
_NW = _NC * _NS
_CHUNK = 128


_BI = 8192
_NMAIN = 61
_H = _BI * _NMAIN


@functools.lru_cache(maxsize=None)
def _make_repack(V, D):
    n_tail = (V - 2 * _H + _BI - 1) // _BI
    grid = _NMAIN + n_tail

    def body(lo_ref, hi_ref, out_ref):
        out_ref[...] = jnp.concatenate(
            [lo_ref[...].T, hi_ref[...].T], axis=1
        )

    return pl.pallas_call(
        body,
        grid=(grid,),
        in_specs=[
            pl.BlockSpec(
                (D, _BI),
                lambda g: (0, jnp.where(g < _NMAIN, g, g + _NMAIN)),
            ),
            pl.BlockSpec((D, _BI), lambda g: (0, g + _NMAIN)),
        ],
        out_specs=pl.BlockSpec((_BI, 2 * D), lambda g: (g, 0)),
        out_shape=jax.ShapeDtypeStruct((grid * _BI, 2 * D), jnp.float32),
    )


@functools.lru_cache(maxsize=None)
def _make_lookup(V, D, B):
    b_per_w = B // _NW
    n_chunks = b_per_w // _CHUNK
    mesh = plsc.VectorSubcoreMesh(core_axis_name="c", subcore_axis_name="s")

    @functools.partial(
        pl.kernel,
        mesh=mesh,
        out_type=jax.ShapeDtypeStruct((B * D,), jnp.float32),
        scratch_types=[
            pltpu.VMEM((b_per_w,), jnp.int32),
            pltpu.VMEM((n_chunks, _CHUNK), jnp.int32),
            pltpu.VMEM((b_per_w, 2 * D), jnp.float32),
            pltpu.VMEM((b_per_w * D,), jnp.float32),
            pltpu.SemaphoreType.DMA,
        ],
    )
    def lookup(packed_hbm, idx_hbm, out_hbm, idx_v, p_v, pairs_v, rows_v, sem):
        wid = lax.axis_index("s") * _NC + lax.axis_index("c")
        base = wid * b_per_w
        pltpu.sync_copy(idx_hbm.at[pl.ds(base, b_per_w)], idx_v)

        for r in range(n_chunks):

            def mkp(g, _, r=r):
                vec = idx_v[pl.ds(r * _CHUNK + g * 16, 16)]
                p_v[r, pl.ds(g * 16, 16)] = jnp.where(vec >= _H, vec - _H, vec)
                return 0

            lax.fori_loop(0, _CHUNK // 16, mkp, 0)

        copies = [
            pltpu.async_copy(
                packed_hbm.at[p_v.at[r]],
                pairs_v.at[pl.ds(r * _CHUNK, _CHUNK)],
                sem,
            )
            for r in range(n_chunks)
        ]
        for c in copies:
            c.wait()

        def sel(g, _):
            vec = idx_v[pl.ds(g * 16, 16)]
            for j in range(16):
                k = g * 16 + j
                half = (vec[j] >= _H).astype(jnp.int32) * D
                for t in range(D // 16):
                    rows_v[pl.ds(k * D + t * 16, 16)] = pairs_v[
                        k, pl.ds(half + t * 16, 16)
                    ]
            return 0

        lax.fori_loop(0, b_per_w // 16, sel, 0)
        pltpu.sync_copy(rows_v, out_hbm.at[pl.ds(base * D, b_per_w * D)])

    return lookup


def kernel(x, age_embedding_weight):
    (B,) = x.shape
    V, D = age_embedding_weight.shape
    wt = age_embedding_weight.T
    packed = _make_repack(V, D)(wt, wt)
    flat = _make_lookup(V, D, B)(packed, x.astype(jnp.int32))
    return flat.reshape(B, D)
